# Initial kernel scaffold; baseline (speedup 1.0000x reference)
import functools

import jax
import jax.numpy as jnp
from jax import lax
from jax.experimental import pallas as pl
from jax.experimental.pallas import tpu as pltpu
from jax.experimental.pallas import tpu_sc as plsc

NC = 2
NS = 16
NW = NC * NS

EMB = 8
CHUNK = 2928


@functools.partial(jax.jit, static_argnames=("n_rows",))
def _sc_gather(idx, table, n_rows):
    per_w = n_rows // NW
    n_chunks = per_w // CHUNK
    assert per_w % CHUNK == 0

    mesh = plsc.VectorSubcoreMesh(core_axis_name="c", subcore_axis_name="s")

    @functools.partial(
        pl.kernel,
        out_type=jax.ShapeDtypeStruct((n_rows, EMB), jnp.float32),
        mesh=mesh,
        scratch_types=[
            pltpu.VMEM((CHUNK,), jnp.int32),
            pltpu.VMEM((CHUNK, EMB), jnp.float32),
            pltpu.SemaphoreType.DMA,
        ],
    )
    def k(idx_hbm, table_hbm, out_hbm, idx_v, rows_v, sem):
        wid = lax.axis_index("s") * NC + lax.axis_index("c")
        base = wid * per_w

        def body(j, carry):
            off = base + j * CHUNK
            pltpu.sync_copy(idx_hbm.at[pl.ds(off, CHUNK)], idx_v)
            pltpu.async_copy(table_hbm.at[idx_v], rows_v, sem).wait()
            pltpu.sync_copy(rows_v, out_hbm.at[pl.ds(off, CHUNK)])
            return carry

        lax.fori_loop(0, n_chunks, body, 0)

    return k(idx, table)


def kernel(grid, item, table):
    B = grid.shape[0]
    idx = jnp.concatenate(
        [grid.reshape(B, -1), item.reshape(B, -1)], axis=1
    ).reshape(-1).astype(jnp.int32)
    out = _sc_gather(idx, table, n_rows=idx.shape[0])
    return out.reshape(B, -1)


# speedup vs baseline: 91.1476x; 91.1476x over previous
import functools

import jax
import jax.numpy as jnp
from jax import lax
from jax.experimental import pallas as pl
from jax.experimental.pallas import tpu as pltpu
from jax.experimental.pallas import tpu_sc as plsc

NC = 2
NS = 16
NW = NC * NS

EMB = 8
CHUNK = 2928


@functools.partial(jax.jit, static_argnames=("n_rows",))
def _sc_gather(idx, table, n_rows):
    per_w = n_rows // NW
    n_chunks = per_w // CHUNK
    assert per_w % CHUNK == 0

    mesh = plsc.VectorSubcoreMesh(core_axis_name="c", subcore_axis_name="s")

    @functools.partial(
        pl.kernel,
        out_type=jax.ShapeDtypeStruct((n_rows, EMB), jnp.float32),
        mesh=mesh,
        scratch_types=[
            pltpu.VMEM((CHUNK,), jnp.int32),
            pltpu.VMEM((CHUNK, EMB), jnp.float32),
            pltpu.SemaphoreType.DMA,
        ],
        compiler_params=pltpu.CompilerParams(use_tc_tiling_on_sc=False),
    )
    def k(idx_hbm, table_hbm, out_hbm, idx_v, rows_v, sem):
        wid = lax.axis_index("s") * NC + lax.axis_index("c")
        base = wid * per_w

        def body(j, carry):
            off = base + j * CHUNK
            pltpu.sync_copy(idx_hbm.at[pl.ds(off, CHUNK)], idx_v)
            pltpu.async_copy(table_hbm.at[idx_v], rows_v, sem).wait()
            pltpu.sync_copy(rows_v, out_hbm.at[pl.ds(off, CHUNK)])
            return carry

        lax.fori_loop(0, n_chunks, body, 0)

    return k(idx, table)


def kernel(grid, item, table):
    B = grid.shape[0]
    idx = jnp.concatenate(
        [grid.reshape(B, -1), item.reshape(B, -1)], axis=1
    ).reshape(-1).astype(jnp.int32)
    out = _sc_gather(idx, table, n_rows=idx.shape[0])
    return out.reshape(B, -1)


# speedup vs baseline: 99.4198x; 1.0908x over previous
import functools

import jax
import jax.numpy as jnp
from jax import lax
from jax.experimental import pallas as pl
from jax.experimental.pallas import tpu as pltpu
from jax.experimental.pallas import tpu_sc as plsc

NC = 2
NS = 16
NW = NC * NS

EMB = 8
CHUNK = 2928


@functools.partial(jax.jit, static_argnames=("n_rows",))
def _sc_gather(idx, table, n_rows):
    per_w = n_rows // NW
    n_chunks = per_w // CHUNK
    assert per_w % CHUNK == 0 and n_chunks >= 2

    mesh = plsc.VectorSubcoreMesh(core_axis_name="c", subcore_axis_name="s")

    @functools.partial(
        pl.kernel,
        out_type=jax.ShapeDtypeStruct((n_rows, EMB), jnp.float32),
        mesh=mesh,
        scratch_types=[
            pltpu.VMEM((CHUNK,), jnp.int32),
            pltpu.VMEM((CHUNK,), jnp.int32),
            pltpu.VMEM((CHUNK, EMB), jnp.float32),
            pltpu.VMEM((CHUNK, EMB), jnp.float32),
            pltpu.SemaphoreType.DMA,
            pltpu.SemaphoreType.DMA,
            pltpu.SemaphoreType.DMA,
            pltpu.SemaphoreType.DMA,
        ],
        compiler_params=pltpu.CompilerParams(use_tc_tiling_on_sc=False),
    )
    def k(idx_hbm, table_hbm, out_hbm, idx0, idx1, rows0, rows1,
          sg0, sg1, sw0, sw1):
        wid = lax.axis_index("s") * NC + lax.axis_index("c")
        base = wid * per_w
        idx_v = (idx0, idx1)
        rows_v = (rows0, rows1)
        sg = (sg0, sg1)
        sw = (sw0, sw1)

        def idx_copy(j, p):
            pltpu.sync_copy(idx_hbm.at[pl.ds(base + j * CHUNK, CHUNK)],
                            idx_v[p])

        def gather_start(j, p):
            return pltpu.async_copy(table_hbm.at[idx_v[p]],
                                    rows_v[p], sg[p])

        def write_start(j, p):
            return pltpu.async_copy(rows_v[p],
                                    out_hbm.at[pl.ds(base + j * CHUNK, CHUNK)],
                                    sw[p])

        idx_copy(0, 0)
        gathers = [gather_start(0, 0), None]
        writes = [None, None]
        for j in range(n_chunks):
            p = j % 2
            q = 1 - p
            if j + 1 < n_chunks:
                idx_copy(j + 1, q)
                if j >= 1:
                    writes[q].wait()
                gathers[q] = gather_start(j + 1, q)
            gathers[p].wait()
            writes[p] = write_start(j, p)
        writes[(n_chunks - 1) % 2].wait()
        writes[(n_chunks - 2) % 2].wait()

    return k(idx, table)


def kernel(grid, item, table):
    B = grid.shape[0]
    idx = jnp.concatenate(
        [grid.reshape(B, -1), item.reshape(B, -1)], axis=1
    ).reshape(-1).astype(jnp.int32)
    out = _sc_gather(idx, table, n_rows=idx.shape[0])
    return out.reshape(B, -1)


# speedup vs baseline: 112.8183x; 1.1348x over previous
import functools

import jax
import jax.numpy as jnp
from jax import lax
from jax.experimental import pallas as pl
from jax.experimental.pallas import tpu as pltpu
from jax.experimental.pallas import tpu_sc as plsc

NC = 2
NS = 16
NW = NC * NS

EMB = 8
LANE = 128
CHUNK = 2928

V = 1000000
VPAD = 1000064
NTILES = VPAD // LANE


@jax.jit
def _sc_detile(t3):
    mesh = plsc.VectorSubcoreMesh(core_axis_name="c", subcore_axis_name="s")
    steps = (NTILES + NW - 1) // NW

    @functools.partial(
        pl.kernel,
        out_type=jax.ShapeDtypeStruct((VPAD * EMB,), jnp.float32),
        mesh=mesh,
        scratch_types=[
            pltpu.VMEM((EMB, LANE), jnp.float32),
            pltpu.VMEM((EMB, LANE), jnp.float32),
            pltpu.VMEM((LANE * EMB,), jnp.float32),
            pltpu.VMEM((LANE * EMB,), jnp.float32),
            pltpu.SemaphoreType.DMA,
            pltpu.SemaphoreType.DMA,
        ],
        compiler_params=pltpu.CompilerParams(
            use_tc_tiling_on_sc=False, needs_layout_passes=False),
    )
    def k(t3_hbm, out_hbm, in0, in1, ot0, ot1, si, so):
        wid = lax.axis_index("s") * NC + lax.axis_index("c")
        ins = (in0, in1)
        outs = (ot0, ot1)
        lane = lax.iota(jnp.int32, 16)
        f_idx = lax.bitwise_and(lane, 7)
        c_base = lax.shift_right_logical(lane, 3)

        def transpose(src, dst):
            for m in range(LANE * EMB // 16):
                c_idx = c_base + (2 * m)
                v = plsc.load_gather(src, [f_idx, c_idx])
                dst[pl.ds(16 * m, 16)] = v

        def step(j, carry):
            t = wid + j * NW

            @pl.when(t < NTILES)
            def _():
                p = 0
                pltpu.sync_copy(t3_hbm.at[t], ins[p])
                transpose(ins[p], outs[p])
                pltpu.sync_copy(outs[p], out_hbm.at[pl.ds(t * LANE * EMB,
                                                          LANE * EMB)])
            return carry

        lax.fori_loop(0, steps, step, 0)

    return k(t3)


@functools.partial(jax.jit, static_argnames=("n_rows",))
def _sc_gather(idx, table, n_rows):
    per_w = n_rows // NW
    n_chunks = per_w // CHUNK
    assert per_w % CHUNK == 0 and n_chunks >= 2

    mesh = plsc.VectorSubcoreMesh(core_axis_name="c", subcore_axis_name="s")

    @functools.partial(
        pl.kernel,
        out_type=jax.ShapeDtypeStruct((n_rows, EMB), jnp.float32),
        mesh=mesh,
        scratch_types=[
            pltpu.VMEM((CHUNK,), jnp.int32),
            pltpu.VMEM((CHUNK,), jnp.int32),
            pltpu.VMEM((CHUNK, EMB), jnp.float32),
            pltpu.VMEM((CHUNK, EMB), jnp.float32),
            pltpu.SemaphoreType.DMA,
            pltpu.SemaphoreType.DMA,
            pltpu.SemaphoreType.DMA,
            pltpu.SemaphoreType.DMA,
        ],
        compiler_params=pltpu.CompilerParams(use_tc_tiling_on_sc=False),
    )
    def k(idx_hbm, table_hbm, out_hbm, idx0, idx1, rows0, rows1,
          sg0, sg1, sw0, sw1):
        wid = lax.axis_index("s") * NC + lax.axis_index("c")
        base = wid * per_w
        idx_v = (idx0, idx1)
        rows_v = (rows0, rows1)
        sg = (sg0, sg1)
        sw = (sw0, sw1)

        def idx_copy(j, p):
            pltpu.sync_copy(idx_hbm.at[pl.ds(base + j * CHUNK, CHUNK)],
                            idx_v[p])

        def gather_start(j, p):
            return pltpu.async_copy(table_hbm.at[idx_v[p]],
                                    rows_v[p], sg[p])

        def write_start(j, p):
            return pltpu.async_copy(rows_v[p],
                                    out_hbm.at[pl.ds(base + j * CHUNK, CHUNK)],
                                    sw[p])

        idx_copy(0, 0)
        gathers = [gather_start(0, 0), None]
        writes = [None, None]
        for j in range(n_chunks):
            p = j % 2
            q = 1 - p
            if j + 1 < n_chunks:
                idx_copy(j + 1, q)
                if j >= 1:
                    writes[q].wait()
                gathers[q] = gather_start(j + 1, q)
            gathers[p].wait()
            writes[p] = write_start(j, p)
        writes[(n_chunks - 1) % 2].wait()
        writes[(n_chunks - 2) % 2].wait()

    return k(idx, table)


def kernel(grid, item, table):
    B = grid.shape[0]
    idx = jnp.concatenate(
        [grid.reshape(B, -1), item.reshape(B, -1)], axis=1
    ).reshape(-1).astype(jnp.int32)
    tpad = jnp.pad(table, ((0, VPAD - V), (0, 0)))
    t3 = tpad.reshape(NTILES, LANE, EMB).swapaxes(1, 2)
    table_rm = _sc_detile(t3).reshape(VPAD, EMB)
    out = _sc_gather(idx, table_rm, n_rows=idx.shape[0])
    return out.reshape(B, -1)


# speedup vs baseline: 130.6176x; 1.1578x over previous
import functools

import jax
import jax.numpy as jnp
from jax import lax
from jax.experimental import pallas as pl
from jax.experimental.pallas import tpu as pltpu
from jax.experimental.pallas import tpu_sc as plsc

NC = 2
NS = 16
NW = NC * NS

EMB = 8
LANE = 128
CHUNK = 2928

V = 1000000
VPAD = 1048576
NTILES = VPAD // LANE
TILE_W = LANE * EMB
TB = 8
NBLK = NTILES // TB


@jax.jit
def _sc_detile(t3flat):
    mesh = plsc.VectorSubcoreMesh(core_axis_name="c", subcore_axis_name="s")
    blk_per_w = NBLK // NW

    @functools.partial(
        pl.kernel,
        out_type=jax.ShapeDtypeStruct((VPAD * EMB,), jnp.float32),
        mesh=mesh,
        scratch_types=[
            pltpu.VMEM((TB * TILE_W,), jnp.float32),
            pltpu.VMEM((TB * TILE_W,), jnp.float32),
            pltpu.VMEM((TB * TILE_W,), jnp.float32),
            pltpu.VMEM((TB * TILE_W,), jnp.float32),
            pltpu.SemaphoreType.DMA,
            pltpu.SemaphoreType.DMA,
            pltpu.SemaphoreType.DMA,
            pltpu.SemaphoreType.DMA,
        ],
        compiler_params=pltpu.CompilerParams(
            use_tc_tiling_on_sc=False, needs_layout_passes=False),
    )
    def k(t3_hbm, out_hbm, in0, in1, ot0, ot1, si0, si1, so0, so1):
        wid = lax.axis_index("s") * NC + lax.axis_index("c")
        ins = (in0, in1)
        outs = (ot0, ot1)
        sis = (si0, si1)
        sos = (so0, so1)
        lane = lax.iota(jnp.int32, 16)
        flat_base = lax.shift_left(lax.bitwise_and(lane, 7), 7) + \
            lax.shift_right_logical(lane, 3)

        def transpose(src, dst):
            def tile_body(tt, carry):
                off = tt * TILE_W
                base_t = flat_base + off
                for m in range(TILE_W // 16):
                    dst[pl.ds(off + 16 * m, 16)] = \
                        plsc.load_gather(src, [base_t + 2 * m])
                return carry
            lax.fori_loop(0, TB, tile_body, 0)

        def body(jj, carry):
            blk0 = wid + (2 * jj) * NW
            blk1 = wid + (2 * jj + 1) * NW
            cin0 = pltpu.async_copy(
                t3_hbm.at[pl.ds(blk0 * TB * TILE_W, TB * TILE_W)], ins[0],
                sis[0])
            cin1 = pltpu.async_copy(
                t3_hbm.at[pl.ds(blk1 * TB * TILE_W, TB * TILE_W)], ins[1],
                sis[1])
            cin0.wait()
            transpose(ins[0], outs[0])
            co0 = pltpu.async_copy(
                outs[0], out_hbm.at[pl.ds(blk0 * TB * TILE_W, TB * TILE_W)],
                sos[0])
            cin1.wait()
            transpose(ins[1], outs[1])
            co1 = pltpu.async_copy(
                outs[1], out_hbm.at[pl.ds(blk1 * TB * TILE_W, TB * TILE_W)],
                sos[1])
            co0.wait()
            co1.wait()
            return carry

        lax.fori_loop(0, blk_per_w // 2, body, 0)

    return k(t3flat)


@functools.partial(jax.jit, static_argnames=("n_rows",))
def _sc_gather(idx, table, n_rows):
    per_w = n_rows // NW
    n_chunks = per_w // CHUNK
    assert per_w % CHUNK == 0 and n_chunks >= 2

    mesh = plsc.VectorSubcoreMesh(core_axis_name="c", subcore_axis_name="s")

    @functools.partial(
        pl.kernel,
        out_type=jax.ShapeDtypeStruct((n_rows, EMB), jnp.float32),
        mesh=mesh,
        scratch_types=[
            pltpu.VMEM((CHUNK,), jnp.int32),
            pltpu.VMEM((CHUNK,), jnp.int32),
            pltpu.VMEM((CHUNK, EMB), jnp.float32),
            pltpu.VMEM((CHUNK, EMB), jnp.float32),
            pltpu.SemaphoreType.DMA,
            pltpu.SemaphoreType.DMA,
            pltpu.SemaphoreType.DMA,
            pltpu.SemaphoreType.DMA,
        ],
        compiler_params=pltpu.CompilerParams(use_tc_tiling_on_sc=False),
    )
    def k(idx_hbm, table_hbm, out_hbm, idx0, idx1, rows0, rows1,
          sg0, sg1, sw0, sw1):
        wid = lax.axis_index("s") * NC + lax.axis_index("c")
        base = wid * per_w
        idx_v = (idx0, idx1)
        rows_v = (rows0, rows1)
        sg = (sg0, sg1)
        sw = (sw0, sw1)

        def idx_copy(j, p):
            pltpu.sync_copy(idx_hbm.at[pl.ds(base + j * CHUNK, CHUNK)],
                            idx_v[p])

        def gather_start(j, p):
            return pltpu.async_copy(table_hbm.at[idx_v[p]],
                                    rows_v[p], sg[p])

        def write_start(j, p):
            return pltpu.async_copy(rows_v[p],
                                    out_hbm.at[pl.ds(base + j * CHUNK, CHUNK)],
                                    sw[p])

        idx_copy(0, 0)
        gathers = [gather_start(0, 0), None]
        writes = [None, None]
        for j in range(n_chunks):
            p = j % 2
            q = 1 - p
            if j + 1 < n_chunks:
                idx_copy(j + 1, q)
                if j >= 1:
                    writes[q].wait()
                gathers[q] = gather_start(j + 1, q)
            gathers[p].wait()
            writes[p] = write_start(j, p)
        writes[(n_chunks - 1) % 2].wait()
        writes[(n_chunks - 2) % 2].wait()

    return k(idx, table)


def kernel(grid, item, table):
    B = grid.shape[0]
    idx = jnp.concatenate(
        [grid.reshape(B, -1), item.reshape(B, -1)], axis=1
    ).reshape(-1).astype(jnp.int32)
    tpad = jnp.pad(table, ((0, VPAD - V), (0, 0)))
    t3flat = tpad.reshape(NTILES, LANE, EMB).swapaxes(1, 2).reshape(-1)
    table_rm = _sc_detile(t3flat).reshape(VPAD, EMB)
    out = _sc_gather(idx, table_rm, n_rows=idx.shape[0])
    return out.reshape(B, -1)
